# baseline (speedup 1.0000x reference)
import functools

import jax
import jax.numpy as jnp
from jax import lax
from jax.experimental import pallas as pl
from jax.experimental.pallas import tpu as pltpu
from jax.experimental.pallas import tpu_sc as plsc

B = 8
N = 480000
S = 48
W = 346
H = 260
SEG = N // S
SIDX = 3
NW = 32
PAIRS_PER_W = (B * S) // NW

PX = 352
PY = 272
STX = 347
STY = 261
AX = 16 * STX + 16
AY = 16 * STY + 16


def _sc_hist_body(ev_hbm, ox_hbm, oy_hbm, buf, subx, suby, obx, oby):
    cid = lax.axis_index("c")
    sid = lax.axis_index("s")
    wid = sid * 2 + cid
    iota = lax.iota(jnp.int32, 16)
    lanex = iota * STX
    laney = iota * STY
    ones = jnp.full((16,), 1.0, jnp.float32)
    zeros = jnp.zeros((16,), jnp.float32)
    col0 = jnp.zeros((16,), jnp.int32)
    col1 = jnp.full((16,), 1, jnp.int32)

    def pair_body(k, carry):
        pair = wid * PAIRS_PER_W + k
        b = pair // S
        s = pair % S
        pltpu.sync_copy(ev_hbm.at[b, pl.ds(s * SEG, SEG), :], buf)

        def zx(i, c):
            subx[pl.ds(i * 16, 16)] = zeros
            return c

        lax.fori_loop(0, AX // 16, zx, 0)

        def zy(i, c):
            suby[pl.ds(i * 16, 16)] = zeros
            return c

        lax.fori_loop(0, AY // 16, zy, 0)

        def scat(nn, c):
            ev = nn * 16 + iota
            xv = plsc.load_gather(buf, [ev, col0])
            yv = plsc.load_gather(buf, [ev, col1])
            xi = xv.astype(jnp.int32) + lanex
            yi = yv.astype(jnp.int32) + laney
            plsc.addupdate_scatter(subx, [xi], ones)
            plsc.addupdate_scatter(suby, [yi], ones)
            return c

        lax.fori_loop(0, SEG // 16, scat, 0)

        def dx(j, c):
            acc = subx[pl.ds(j * 16, 16)]
            for l in range(1, 16):
                acc = acc + subx[pl.ds(l * STX + j * 16, 16)]
            obx[pl.ds(j * 16, 16)] = acc
            return c

        lax.fori_loop(0, PX // 16, dx, 0)

        def dy(j, c):
            acc = suby[pl.ds(j * 16, 16)]
            for l in range(1, 16):
                acc = acc + suby[pl.ds(l * STY + j * 16, 16)]
            oby[pl.ds(j * 16, 16)] = acc
            return c

        lax.fori_loop(0, PY // 16, dy, 0)

        pltpu.sync_copy(obx, ox_hbm.at[b, s, :])
        pltpu.sync_copy(oby, oy_hbm.at[b, s, :])
        return carry

    lax.fori_loop(0, PAIRS_PER_W, pair_body, 0)


@functools.lru_cache(maxsize=None)
def _build_sc_hist():
    return functools.partial(
        pl.kernel,
        mesh=plsc.VectorSubcoreMesh(core_axis_name="c", subcore_axis_name="s"),
        compiler_params=pltpu.CompilerParams(
            use_tc_tiling_on_sc=False, needs_layout_passes=False),
        out_type=[
            jax.ShapeDtypeStruct((B, S, PX), jnp.float32),
            jax.ShapeDtypeStruct((B, S, PY), jnp.float32),
        ],
        scratch_types=[
            pltpu.VMEM((SEG, 5), jnp.float32),
            pltpu.VMEM((AX,), jnp.float32),
            pltpu.VMEM((AY,), jnp.float32),
            pltpu.VMEM((PX,), jnp.float32),
            pltpu.VMEM((PY,), jnp.float32),
        ],
    )(_sc_hist_body)


def _aligned_shift(hist, D, P):
    n = float(S * D)
    dP = lax.broadcasted_iota(jnp.int32, (B, S, P), 2)
    h = jnp.where(dP < D, hist, 0.0)
    sm = jnp.sum(h, axis=(1, 2))
    sq = jnp.sum(h * h, axis=(1, 2))
    mean = sm / n
    var = (sq - sm * sm / n) / (n - 1.0)
    clamp = mean + 3.0 * jnp.sqrt(var)
    rows = h[:, SIDX - 2:SIDX + 3, :]
    rows = jnp.clip(rows, 0.0, clamp[:, None, None])
    rowsum = jnp.sum(rows, axis=1)
    di = lax.broadcasted_iota(jnp.int32, (B, P), 1)
    w = 5.0 * di.astype(jnp.float32)
    w = jnp.where(di == 0, 3.0, w)
    w = jnp.where(di == 1, 6.0, w)
    w = jnp.where(di == D - 2, float(4 * D - 10), w)
    w = jnp.where(di == D - 1, float(3 * D - 6), w)
    w = jnp.where(di >= D, 0.0, w)
    meanD = jnp.sum(rowsum * w, axis=1) * (0.04 / float(SEG))
    return jnp.round(meanD - float(D // 2))


def _post_body(hx_ref, hy_ref, x_ref, y_ref, t_ref, o_ref):
    shx = _aligned_shift(hx_ref[...], W, PX)
    shy = _aligned_shift(hy_ref[...], H, PY)
    xv = jnp.clip(x_ref[...] - shx[:, None], 0.0, float(W - 1)) * (1.0 / W)
    yv = jnp.clip(y_ref[...] - shy[:, None], 0.0, float(H - 1)) * (1.0 / H)
    t = t_ref[...]
    tv = t / jnp.max(t, axis=1, keepdims=True)
    o_ref[...] = jnp.stack([xv, yv, tv], axis=1)


_tc_post = pl.pallas_call(
    _post_body,
    out_shape=jax.ShapeDtypeStruct((B, 3, 2048), jnp.float32),
)


@jax.jit
def kernel(events):
    hx, hy = _build_sc_hist()(events)
    first = SEG * SIDX
    sl = lax.slice(events, (0, first, 0), (B, first + 2048, 3))
    x_sl = sl[:, :, 0]
    y_sl = sl[:, :, 1]
    t_sl = sl[:, :, 2]
    return _tc_post(hx, hy, x_sl, y_sl, t_sl)


# speedup vs baseline: 14.4376x; 14.4376x over previous
import functools

import jax
import jax.numpy as jnp
from jax import lax
from jax.experimental import pallas as pl
from jax.experimental.pallas import tpu as pltpu
from jax.experimental.pallas import tpu_sc as plsc

B = 8
N = 480000
S = 48
W = 346
H = 260
SEG = N // S
SIDX = 3
NW = 32
PAIRS_PER_W = (B * S) // NW

PX = 352
PY = 272
STX = 347
STY = 261
AX = 16 * STX + 16
AY = 16 * STY + 16


def _sc_hist_body(xi_hbm, yi_hbm, ox_hbm, oy_hbm, bufx, bufy, subx, suby,
                  obx, oby):
    cid = lax.axis_index("c")
    sid = lax.axis_index("s")
    wid = sid * 2 + cid
    iota = lax.iota(jnp.int32, 16)
    lanex = iota * STX
    laney = iota * STY
    ones = jnp.full((16,), 1.0, jnp.float32)
    zeros = jnp.zeros((16,), jnp.float32)

    def pair_body(k, carry):
        pair = wid * PAIRS_PER_W + k
        b = pair // S
        s = pair % S
        pltpu.sync_copy(xi_hbm.at[pl.ds(b * N + s * SEG, SEG)], bufx)
        pltpu.sync_copy(yi_hbm.at[pl.ds(b * N + s * SEG, SEG)], bufy)

        def zx(i, c):
            subx[pl.ds(i * 16, 16)] = zeros
            return c

        lax.fori_loop(0, AX // 16, zx, 0)

        def zy(i, c):
            suby[pl.ds(i * 16, 16)] = zeros
            return c

        lax.fori_loop(0, AY // 16, zy, 0)

        def scat(nn, c):
            xv = bufx[pl.ds(nn * 16, 16)] + lanex
            yv = bufy[pl.ds(nn * 16, 16)] + laney
            plsc.addupdate_scatter(subx, [xv], ones)
            plsc.addupdate_scatter(suby, [yv], ones)
            return c

        lax.fori_loop(0, SEG // 16, scat, 0)

        def dx(j, c):
            acc = subx[pl.ds(j * 16, 16)]
            for l in range(1, 16):
                acc = acc + subx[pl.ds(l * STX + j * 16, 16)]
            obx[pl.ds(j * 16, 16)] = acc
            return c

        lax.fori_loop(0, PX // 16, dx, 0)

        def dy(j, c):
            acc = suby[pl.ds(j * 16, 16)]
            for l in range(1, 16):
                acc = acc + suby[pl.ds(l * STY + j * 16, 16)]
            oby[pl.ds(j * 16, 16)] = acc
            return c

        lax.fori_loop(0, PY // 16, dy, 0)

        pltpu.sync_copy(obx, ox_hbm.at[b, s, :])
        pltpu.sync_copy(oby, oy_hbm.at[b, s, :])
        return carry

    lax.fori_loop(0, PAIRS_PER_W, pair_body, 0)


@functools.lru_cache(maxsize=None)
def _build_sc_hist():
    return functools.partial(
        pl.kernel,
        mesh=plsc.VectorSubcoreMesh(core_axis_name="c", subcore_axis_name="s"),
        compiler_params=pltpu.CompilerParams(
            use_tc_tiling_on_sc=False, needs_layout_passes=False),
        out_type=[
            jax.ShapeDtypeStruct((B, S, PX), jnp.float32),
            jax.ShapeDtypeStruct((B, S, PY), jnp.float32),
        ],
        scratch_types=[
            pltpu.VMEM((SEG,), jnp.int32),
            pltpu.VMEM((SEG,), jnp.int32),
            pltpu.VMEM((AX,), jnp.float32),
            pltpu.VMEM((AY,), jnp.float32),
            pltpu.VMEM((PX,), jnp.float32),
            pltpu.VMEM((PY,), jnp.float32),
        ],
    )(_sc_hist_body)


def _aligned_shift(hist, D, P):
    n = float(S * D)
    dP = lax.broadcasted_iota(jnp.int32, (B, S, P), 2)
    h = jnp.where(dP < D, hist, 0.0)
    sm = jnp.sum(h, axis=(1, 2))
    sq = jnp.sum(h * h, axis=(1, 2))
    mean = sm / n
    var = (sq - sm * sm / n) / (n - 1.0)
    clamp = mean + 3.0 * jnp.sqrt(var)
    rows = h[:, SIDX - 2:SIDX + 3, :]
    rows = jnp.clip(rows, 0.0, clamp[:, None, None])
    rowsum = jnp.sum(rows, axis=1)
    di = lax.broadcasted_iota(jnp.int32, (B, P), 1)
    w = 5.0 * di.astype(jnp.float32)
    w = jnp.where(di == 0, 3.0, w)
    w = jnp.where(di == 1, 6.0, w)
    w = jnp.where(di == D - 2, float(4 * D - 10), w)
    w = jnp.where(di == D - 1, float(3 * D - 6), w)
    w = jnp.where(di >= D, 0.0, w)
    meanD = jnp.sum(rowsum * w, axis=1) * (0.04 / float(SEG))
    return jnp.round(meanD - float(D // 2))


def _post_body(hx_ref, hy_ref, x_ref, y_ref, t_ref, o_ref):
    shx = _aligned_shift(hx_ref[...], W, PX)
    shy = _aligned_shift(hy_ref[...], H, PY)
    xv = jnp.clip(x_ref[...] - shx[:, None], 0.0, float(W - 1)) * (1.0 / W)
    yv = jnp.clip(y_ref[...] - shy[:, None], 0.0, float(H - 1)) * (1.0 / H)
    t = t_ref[...]
    tv = t / jnp.max(t, axis=1, keepdims=True)
    o_ref[...] = jnp.stack([xv, yv, tv], axis=1)


_tc_post = pl.pallas_call(
    _post_body,
    out_shape=jax.ShapeDtypeStruct((B, 3, 2048), jnp.float32),
)


@jax.jit
def kernel(events):
    xi1 = events[:, :, 0].astype(jnp.int32).reshape(B * N)
    yi1 = events[:, :, 1].astype(jnp.int32).reshape(B * N)
    hx, hy = _build_sc_hist()(xi1, yi1)
    first = SEG * SIDX
    sl = lax.slice(events, (0, first, 0), (B, first + 2048, 3))
    x_sl = sl[:, :, 0]
    y_sl = sl[:, :, 1]
    t_sl = sl[:, :, 2]
    return _tc_post(hx, hy, x_sl, y_sl, t_sl)


# speedup vs baseline: 18.9108x; 1.3098x over previous
import functools

import jax
import jax.numpy as jnp
from jax import lax
from jax.experimental import pallas as pl
from jax.experimental.pallas import tpu as pltpu
from jax.experimental.pallas import tpu_sc as plsc

B = 8
N = 480000
S = 48
W = 346
H = 260
SEG = N // S
SIDX = 3
NW = 32
PAIRS_PER_W = (B * S) // NW

PX = 352
PY = 272
STX = 347
STY = 261
AX = 16 * STX + 16
AY = 16 * STY + 16


def _sc_hist_body(xi_hbm, yi_hbm, ox_hbm, oy_hbm, bufx, bufy, obx, oby):
    cid = lax.axis_index("c")
    sid = lax.axis_index("s")
    wid = sid * 2 + cid
    ones = jnp.full((16,), 1.0, jnp.float32)
    zeros = jnp.zeros((16,), jnp.float32)

    def pair_body(k, carry):
        pair = wid * PAIRS_PER_W + k
        b = pair // S
        s = pair % S
        pltpu.sync_copy(xi_hbm.at[pl.ds(b * N + s * SEG, SEG)], bufx)
        pltpu.sync_copy(yi_hbm.at[pl.ds(b * N + s * SEG, SEG)], bufy)

        for j in range(PX // 16):
            obx[pl.ds(j * 16, 16)] = zeros
        for j in range(PY // 16):
            oby[pl.ds(j * 16, 16)] = zeros

        def scat(nn, c):
            xv = bufx[pl.ds(nn * 16, 16)]
            yv = bufy[pl.ds(nn * 16, 16)]
            plsc.addupdate_scatter(obx, [xv], ones)
            plsc.addupdate_scatter(oby, [yv], ones)
            return c

        lax.fori_loop(0, SEG // 16, scat, 0)

        pltpu.sync_copy(obx, ox_hbm.at[b, s, :])
        pltpu.sync_copy(oby, oy_hbm.at[b, s, :])
        return carry

    lax.fori_loop(0, PAIRS_PER_W, pair_body, 0)


@functools.lru_cache(maxsize=None)
def _build_sc_hist():
    return functools.partial(
        pl.kernel,
        mesh=plsc.VectorSubcoreMesh(core_axis_name="c", subcore_axis_name="s"),
        compiler_params=pltpu.CompilerParams(
            use_tc_tiling_on_sc=False, needs_layout_passes=False),
        out_type=[
            jax.ShapeDtypeStruct((B, S, PX), jnp.float32),
            jax.ShapeDtypeStruct((B, S, PY), jnp.float32),
        ],
        scratch_types=[
            pltpu.VMEM((SEG,), jnp.int32),
            pltpu.VMEM((SEG,), jnp.int32),
            pltpu.VMEM((PX,), jnp.float32),
            pltpu.VMEM((PY,), jnp.float32),
        ],
    )(_sc_hist_body)


def _aligned_shift(hist, D, P):
    n = float(S * D)
    dP = lax.broadcasted_iota(jnp.int32, (B, S, P), 2)
    h = jnp.where(dP < D, hist, 0.0)
    sm = jnp.sum(h, axis=(1, 2))
    sq = jnp.sum(h * h, axis=(1, 2))
    mean = sm / n
    var = (sq - sm * sm / n) / (n - 1.0)
    clamp = mean + 3.0 * jnp.sqrt(var)
    rows = h[:, SIDX - 2:SIDX + 3, :]
    rows = jnp.clip(rows, 0.0, clamp[:, None, None])
    rowsum = jnp.sum(rows, axis=1)
    di = lax.broadcasted_iota(jnp.int32, (B, P), 1)
    w = 5.0 * di.astype(jnp.float32)
    w = jnp.where(di == 0, 3.0, w)
    w = jnp.where(di == 1, 6.0, w)
    w = jnp.where(di == D - 2, float(4 * D - 10), w)
    w = jnp.where(di == D - 1, float(3 * D - 6), w)
    w = jnp.where(di >= D, 0.0, w)
    meanD = jnp.sum(rowsum * w, axis=1) * (0.04 / float(SEG))
    return jnp.round(meanD - float(D // 2))


def _post_body(hx_ref, hy_ref, x_ref, y_ref, t_ref, o_ref):
    shx = _aligned_shift(hx_ref[...], W, PX)
    shy = _aligned_shift(hy_ref[...], H, PY)
    xv = jnp.clip(x_ref[...] - shx[:, None], 0.0, float(W - 1)) * (1.0 / W)
    yv = jnp.clip(y_ref[...] - shy[:, None], 0.0, float(H - 1)) * (1.0 / H)
    t = t_ref[...]
    tv = t / jnp.max(t, axis=1, keepdims=True)
    o_ref[...] = jnp.stack([xv, yv, tv], axis=1)


_tc_post = pl.pallas_call(
    _post_body,
    out_shape=jax.ShapeDtypeStruct((B, 3, 2048), jnp.float32),
)


@jax.jit
def kernel(events):
    xi1 = events[:, :, 0].astype(jnp.int32).reshape(B * N)
    yi1 = events[:, :, 1].astype(jnp.int32).reshape(B * N)
    hx, hy = _build_sc_hist()(xi1, yi1)
    first = SEG * SIDX
    sl = lax.slice(events, (0, first, 0), (B, first + 2048, 3))
    x_sl = sl[:, :, 0]
    y_sl = sl[:, :, 1]
    t_sl = sl[:, :, 2]
    return _tc_post(hx, hy, x_sl, y_sl, t_sl)


# speedup vs baseline: 21.7498x; 1.1501x over previous
import functools

import jax
import jax.numpy as jnp
from jax import lax
from jax.experimental import pallas as pl
from jax.experimental.pallas import tpu as pltpu
from jax.experimental.pallas import tpu_sc as plsc

B = 8
N = 480000
S = 48
W = 346
H = 260
SEG = N // S
SIDX = 3
NW = 32
PAIRS_PER_W = (B * S) // NW

PX = 352
PY = 272
STX = 347
STY = 261
AX = 16 * STX + 16
AY = 16 * STY + 16


def _sc_hist_body(code_hbm, ox_hbm, oy_hbm, buf, obx, oby):
    cid = lax.axis_index("c")
    sid = lax.axis_index("s")
    wid = sid * 2 + cid
    ones = jnp.full((16,), 1.0, jnp.float32)
    zeros = jnp.zeros((16,), jnp.float32)
    mask9 = jnp.full((16,), 511, jnp.int32)

    def pair_body(k, carry):
        pair = wid * PAIRS_PER_W + k
        b = pair // S
        s = pair % S
        pltpu.sync_copy(code_hbm.at[pl.ds(b * N + s * SEG, SEG)], buf)

        for j in range(PX // 16):
            obx[pl.ds(j * 16, 16)] = zeros
        for j in range(PY // 16):
            oby[pl.ds(j * 16, 16)] = zeros

        def scat(nn, c):
            code = buf[pl.ds(nn * 16, 16)]
            xv = lax.bitwise_and(code, mask9)
            yv = lax.shift_right_logical(code, 9)
            plsc.addupdate_scatter(obx, [xv], ones)
            plsc.addupdate_scatter(oby, [yv], ones)
            return c

        lax.fori_loop(0, SEG // 16, scat, 0, unroll=5)

        pltpu.sync_copy(obx, ox_hbm.at[b, s, :])
        pltpu.sync_copy(oby, oy_hbm.at[b, s, :])
        return carry

    lax.fori_loop(0, PAIRS_PER_W, pair_body, 0)


@functools.lru_cache(maxsize=None)
def _build_sc_hist():
    return functools.partial(
        pl.kernel,
        mesh=plsc.VectorSubcoreMesh(core_axis_name="c", subcore_axis_name="s"),
        compiler_params=pltpu.CompilerParams(
            use_tc_tiling_on_sc=False, needs_layout_passes=False),
        out_type=[
            jax.ShapeDtypeStruct((B, S, PX), jnp.float32),
            jax.ShapeDtypeStruct((B, S, PY), jnp.float32),
        ],
        scratch_types=[
            pltpu.VMEM((SEG,), jnp.int32),
            pltpu.VMEM((PX,), jnp.float32),
            pltpu.VMEM((PY,), jnp.float32),
        ],
    )(_sc_hist_body)


def _aligned_shift(hist, D, P):
    n = float(S * D)
    dP = lax.broadcasted_iota(jnp.int32, (B, S, P), 2)
    h = jnp.where(dP < D, hist, 0.0)
    sm = jnp.sum(h, axis=(1, 2))
    sq = jnp.sum(h * h, axis=(1, 2))
    mean = sm / n
    var = (sq - sm * sm / n) / (n - 1.0)
    clamp = mean + 3.0 * jnp.sqrt(var)
    rows = h[:, SIDX - 2:SIDX + 3, :]
    rows = jnp.clip(rows, 0.0, clamp[:, None, None])
    rowsum = jnp.sum(rows, axis=1)
    di = lax.broadcasted_iota(jnp.int32, (B, P), 1)
    w = 5.0 * di.astype(jnp.float32)
    w = jnp.where(di == 0, 3.0, w)
    w = jnp.where(di == 1, 6.0, w)
    w = jnp.where(di == D - 2, float(4 * D - 10), w)
    w = jnp.where(di == D - 1, float(3 * D - 6), w)
    w = jnp.where(di >= D, 0.0, w)
    meanD = jnp.sum(rowsum * w, axis=1) * (0.04 / float(SEG))
    return jnp.round(meanD - float(D // 2))


def _post_body(hx_ref, hy_ref, x_ref, y_ref, t_ref, o_ref):
    shx = _aligned_shift(hx_ref[...], W, PX)
    shy = _aligned_shift(hy_ref[...], H, PY)
    xv = jnp.clip(x_ref[...] - shx[:, None], 0.0, float(W - 1)) * (1.0 / W)
    yv = jnp.clip(y_ref[...] - shy[:, None], 0.0, float(H - 1)) * (1.0 / H)
    t = t_ref[...]
    tv = t / jnp.max(t, axis=1, keepdims=True)
    o_ref[...] = jnp.stack([xv, yv, tv], axis=1)


_tc_post = pl.pallas_call(
    _post_body,
    out_shape=jax.ShapeDtypeStruct((B, 3, 2048), jnp.float32),
)


@jax.jit
def kernel(events):
    xi = events[:, :, 0].astype(jnp.int32)
    yi = events[:, :, 1].astype(jnp.int32)
    code = ((yi << 9) | xi).reshape(B * N)
    hx, hy = _build_sc_hist()(code)
    first = SEG * SIDX
    sl = lax.slice(events, (0, first, 0), (B, first + 2048, 3))
    x_sl = sl[:, :, 0]
    y_sl = sl[:, :, 1]
    t_sl = sl[:, :, 2]
    return _tc_post(hx, hy, x_sl, y_sl, t_sl)


# speedup vs baseline: 24.2126x; 1.1132x over previous
import functools

import jax
import jax.numpy as jnp
from jax import lax
from jax.experimental import pallas as pl
from jax.experimental.pallas import tpu as pltpu
from jax.experimental.pallas import tpu_sc as plsc

B = 8
N = 480000
S = 48
W = 346
H = 260
SEG = N // S
SIDX = 3
NW = 32
PAIRS_PER_W = (B * S) // NW

PX = 352
PY = 272
STX = 347
STY = 261
AX = 16 * STX + 16
AY = 16 * STY + 16


def _sc_hist_body(code_hbm, ox_hbm, oy_hbm, buf0, buf1, obx0, oby0, obx1,
                  oby1, isem0, isem1, osem0, osem1):
    cid = lax.axis_index("c")
    sid = lax.axis_index("s")
    wid = sid * 2 + cid
    ones = jnp.full((16,), 1.0, jnp.float32)
    zeros = jnp.zeros((16,), jnp.float32)
    mask9 = jnp.full((16,), 511, jnp.int32)

    def src_at(k):
        pair = wid * PAIRS_PER_W + k
        b = pair // S
        s = pair % S
        return code_hbm.at[pl.ds(b * N + s * SEG, SEG)], b, s

    def fire(k, dst, sem):
        src, _, _ = src_at(k)
        pltpu.async_copy(src, dst, sem)

    def drain_in(dst, sem):
        pltpu.make_async_copy(code_hbm.at[pl.ds(0, SEG)], dst, sem).wait()

    def compute(k, src, obx, oby, osem):
        _, b, s = src_at(k)

        @pl.when(k >= 2)
        def _():
            pltpu.make_async_copy(obx, ox_hbm.at[0, 0, :], osem).wait()
            pltpu.make_async_copy(oby, oy_hbm.at[0, 0, :], osem).wait()

        for j in range(PX // 16):
            obx[pl.ds(j * 16, 16)] = zeros
        for j in range(PY // 16):
            oby[pl.ds(j * 16, 16)] = zeros

        def scat(nn, c):
            code = src[pl.ds(nn * 16, 16)]
            xv = lax.bitwise_and(code, mask9)
            yv = lax.shift_right_logical(code, 9)
            plsc.addupdate_scatter(obx, [xv], ones)
            plsc.addupdate_scatter(oby, [yv], ones)
            return c

        lax.fori_loop(0, SEG // 16, scat, 0, unroll=5)

        pltpu.async_copy(obx, ox_hbm.at[b, s, :], osem)
        pltpu.async_copy(oby, oy_hbm.at[b, s, :], osem)

    fire(0, buf0, isem0)

    def loop(k2, carry):
        k = k2 * 2
        drain_in(buf0, isem0)
        fire(k + 1, buf1, isem1)
        compute(k, buf0, obx0, oby0, osem0)
        drain_in(buf1, isem1)

        @pl.when(k + 2 < PAIRS_PER_W)
        def _():
            fire(k + 2, buf0, isem0)

        compute(k + 1, buf1, obx1, oby1, osem1)
        return carry

    lax.fori_loop(0, PAIRS_PER_W // 2, loop, 0)

    pltpu.make_async_copy(obx0, ox_hbm.at[0, 0, :], osem0).wait()
    pltpu.make_async_copy(oby0, oy_hbm.at[0, 0, :], osem0).wait()
    pltpu.make_async_copy(obx1, ox_hbm.at[0, 0, :], osem1).wait()
    pltpu.make_async_copy(oby1, oy_hbm.at[0, 0, :], osem1).wait()


@functools.lru_cache(maxsize=None)
def _build_sc_hist():
    return functools.partial(
        pl.kernel,
        mesh=plsc.VectorSubcoreMesh(core_axis_name="c", subcore_axis_name="s"),
        compiler_params=pltpu.CompilerParams(
            use_tc_tiling_on_sc=False, needs_layout_passes=False),
        out_type=[
            jax.ShapeDtypeStruct((B, S, PX), jnp.float32),
            jax.ShapeDtypeStruct((B, S, PY), jnp.float32),
        ],
        scratch_types=[
            pltpu.VMEM((SEG,), jnp.int32),
            pltpu.VMEM((SEG,), jnp.int32),
            pltpu.VMEM((PX,), jnp.float32),
            pltpu.VMEM((PY,), jnp.float32),
            pltpu.VMEM((PX,), jnp.float32),
            pltpu.VMEM((PY,), jnp.float32),
            pltpu.SemaphoreType.DMA,
            pltpu.SemaphoreType.DMA,
            pltpu.SemaphoreType.DMA,
            pltpu.SemaphoreType.DMA,
        ],
    )(_sc_hist_body)


def _aligned_shift(hist, D, P):
    n = float(S * D)
    dP = lax.broadcasted_iota(jnp.int32, (B, S, P), 2)
    h = jnp.where(dP < D, hist, 0.0)
    sm = jnp.sum(h, axis=(1, 2))
    sq = jnp.sum(h * h, axis=(1, 2))
    mean = sm / n
    var = (sq - sm * sm / n) / (n - 1.0)
    clamp = mean + 3.0 * jnp.sqrt(var)
    rows = h[:, SIDX - 2:SIDX + 3, :]
    rows = jnp.clip(rows, 0.0, clamp[:, None, None])
    rowsum = jnp.sum(rows, axis=1)
    di = lax.broadcasted_iota(jnp.int32, (B, P), 1)
    w = 5.0 * di.astype(jnp.float32)
    w = jnp.where(di == 0, 3.0, w)
    w = jnp.where(di == 1, 6.0, w)
    w = jnp.where(di == D - 2, float(4 * D - 10), w)
    w = jnp.where(di == D - 1, float(3 * D - 6), w)
    w = jnp.where(di >= D, 0.0, w)
    meanD = jnp.sum(rowsum * w, axis=1) * (0.04 / float(SEG))
    return jnp.round(meanD - float(D // 2))


def _post_body(hx_ref, hy_ref, x_ref, y_ref, t_ref, o_ref):
    shx = _aligned_shift(hx_ref[...], W, PX)
    shy = _aligned_shift(hy_ref[...], H, PY)
    xv = jnp.clip(x_ref[...] - shx[:, None], 0.0, float(W - 1)) * (1.0 / W)
    yv = jnp.clip(y_ref[...] - shy[:, None], 0.0, float(H - 1)) * (1.0 / H)
    t = t_ref[...]
    tv = t / jnp.max(t, axis=1, keepdims=True)
    o_ref[...] = jnp.stack([xv, yv, tv], axis=1)


_tc_post = pl.pallas_call(
    _post_body,
    out_shape=jax.ShapeDtypeStruct((B, 3, 2048), jnp.float32),
)


@jax.jit
def kernel(events):
    xi = events[:, :, 0].astype(jnp.int32)
    yi = events[:, :, 1].astype(jnp.int32)
    code = ((yi << 9) | xi).reshape(B * N)
    hx, hy = _build_sc_hist()(code)
    first = SEG * SIDX
    sl = lax.slice(events, (0, first, 0), (B, first + 2048, 3))
    x_sl = sl[:, :, 0]
    y_sl = sl[:, :, 1]
    t_sl = sl[:, :, 2]
    return _tc_post(hx, hy, x_sl, y_sl, t_sl)


# speedup vs baseline: 36.0559x; 1.4891x over previous
import functools

import jax
import jax.numpy as jnp
from jax import lax
from jax.experimental import pallas as pl
from jax.experimental.pallas import tpu as pltpu
from jax.experimental.pallas import tpu_sc as plsc

B = 8
N = 480000
S = 48
W = 346
H = 260
SEG = N // S
SIDX = 3
NW = 32
PAIRS_PER_W = (B * S) // NW

PX = 352
PY = 272
STX = 347
STY = 261
AX = 16 * STX + 16
AY = 16 * STY + 16


def _sc_hist_body(code_hbm, ox_hbm, oy_hbm, buf0, buf1, obx0, oby0, obx1,
                  oby1, isem0, isem1, osem0, osem1):
    cid = lax.axis_index("c")
    sid = lax.axis_index("s")
    wid = sid * 2 + cid
    ones = jnp.full((16,), 1.0, jnp.float32)
    zeros = jnp.zeros((16,), jnp.float32)
    mask9 = jnp.full((16,), 511, jnp.int32)

    def src_at(k):
        pair = wid * PAIRS_PER_W + k
        b = pair // S
        s = pair % S
        return code_hbm.at[pl.ds(b * N + s * SEG, SEG)], b, s

    def fire(k, dst, sem):
        src, _, _ = src_at(k)
        pltpu.async_copy(src, dst, sem)

    def drain_in(dst, sem):
        pltpu.make_async_copy(code_hbm.at[pl.ds(0, SEG)], dst, sem).wait()

    def compute(k, src, obx, oby, osem):
        _, b, s = src_at(k)

        @pl.when(k >= 2)
        def _():
            pltpu.make_async_copy(obx, ox_hbm.at[0, 0, :], osem).wait()
            pltpu.make_async_copy(oby, oy_hbm.at[0, 0, :], osem).wait()

        for j in range(PX // 16):
            obx[pl.ds(j * 16, 16)] = zeros
        for j in range(PY // 16):
            oby[pl.ds(j * 16, 16)] = zeros

        @plsc.parallel_loop(0, SEG, step=16, unroll=5)
        def scat(nn):
            code = src[pl.ds(nn, 16)]
            xv = lax.bitwise_and(code, mask9)
            yv = lax.shift_right_logical(code, 9)
            plsc.addupdate_scatter(obx, [xv], ones)
            plsc.addupdate_scatter(oby, [yv], ones)

        pltpu.async_copy(obx, ox_hbm.at[b, s, :], osem)
        pltpu.async_copy(oby, oy_hbm.at[b, s, :], osem)

    fire(0, buf0, isem0)

    def loop(k2, carry):
        k = k2 * 2
        drain_in(buf0, isem0)
        fire(k + 1, buf1, isem1)
        compute(k, buf0, obx0, oby0, osem0)
        drain_in(buf1, isem1)

        @pl.when(k + 2 < PAIRS_PER_W)
        def _():
            fire(k + 2, buf0, isem0)

        compute(k + 1, buf1, obx1, oby1, osem1)
        return carry

    lax.fori_loop(0, PAIRS_PER_W // 2, loop, 0)

    pltpu.make_async_copy(obx0, ox_hbm.at[0, 0, :], osem0).wait()
    pltpu.make_async_copy(oby0, oy_hbm.at[0, 0, :], osem0).wait()
    pltpu.make_async_copy(obx1, ox_hbm.at[0, 0, :], osem1).wait()
    pltpu.make_async_copy(oby1, oy_hbm.at[0, 0, :], osem1).wait()


@functools.lru_cache(maxsize=None)
def _build_sc_hist():
    return functools.partial(
        pl.kernel,
        mesh=plsc.VectorSubcoreMesh(core_axis_name="c", subcore_axis_name="s"),
        compiler_params=pltpu.CompilerParams(
            use_tc_tiling_on_sc=False, needs_layout_passes=False),
        out_type=[
            jax.ShapeDtypeStruct((B, S, PX), jnp.float32),
            jax.ShapeDtypeStruct((B, S, PY), jnp.float32),
        ],
        scratch_types=[
            pltpu.VMEM((SEG,), jnp.int32),
            pltpu.VMEM((SEG,), jnp.int32),
            pltpu.VMEM((PX,), jnp.float32),
            pltpu.VMEM((PY,), jnp.float32),
            pltpu.VMEM((PX,), jnp.float32),
            pltpu.VMEM((PY,), jnp.float32),
            pltpu.SemaphoreType.DMA,
            pltpu.SemaphoreType.DMA,
            pltpu.SemaphoreType.DMA,
            pltpu.SemaphoreType.DMA,
        ],
    )(_sc_hist_body)


def _aligned_shift(hist, D, P):
    n = float(S * D)
    dP = lax.broadcasted_iota(jnp.int32, (B, S, P), 2)
    h = jnp.where(dP < D, hist, 0.0)
    sm = jnp.sum(h, axis=(1, 2))
    sq = jnp.sum(h * h, axis=(1, 2))
    mean = sm / n
    var = (sq - sm * sm / n) / (n - 1.0)
    clamp = mean + 3.0 * jnp.sqrt(var)
    rows = h[:, SIDX - 2:SIDX + 3, :]
    rows = jnp.clip(rows, 0.0, clamp[:, None, None])
    rowsum = jnp.sum(rows, axis=1)
    di = lax.broadcasted_iota(jnp.int32, (B, P), 1)
    w = 5.0 * di.astype(jnp.float32)
    w = jnp.where(di == 0, 3.0, w)
    w = jnp.where(di == 1, 6.0, w)
    w = jnp.where(di == D - 2, float(4 * D - 10), w)
    w = jnp.where(di == D - 1, float(3 * D - 6), w)
    w = jnp.where(di >= D, 0.0, w)
    meanD = jnp.sum(rowsum * w, axis=1) * (0.04 / float(SEG))
    return jnp.round(meanD - float(D // 2))


def _post_body(hx_ref, hy_ref, x_ref, y_ref, t_ref, o_ref):
    shx = _aligned_shift(hx_ref[...], W, PX)
    shy = _aligned_shift(hy_ref[...], H, PY)
    xv = jnp.clip(x_ref[...] - shx[:, None], 0.0, float(W - 1)) * (1.0 / W)
    yv = jnp.clip(y_ref[...] - shy[:, None], 0.0, float(H - 1)) * (1.0 / H)
    t = t_ref[...]
    tv = t / jnp.max(t, axis=1, keepdims=True)
    o_ref[...] = jnp.stack([xv, yv, tv], axis=1)


_tc_post = pl.pallas_call(
    _post_body,
    out_shape=jax.ShapeDtypeStruct((B, 3, 2048), jnp.float32),
)


@jax.jit
def kernel(events):
    xi = events[:, :, 0].reshape(B * N).astype(jnp.int32)
    yi = events[:, :, 1].reshape(B * N).astype(jnp.int32)
    code = (yi << 9) | xi
    hx, hy = _build_sc_hist()(code)
    first = SEG * SIDX
    sl = lax.slice(events, (0, first, 0), (B, first + 2048, 3))
    x_sl = sl[:, :, 0]
    y_sl = sl[:, :, 1]
    t_sl = sl[:, :, 2]
    return _tc_post(hx, hy, x_sl, y_sl, t_sl)
